# Initial kernel scaffold; baseline (speedup 1.0000x reference)
import jax
import jax.numpy as jnp
from jax.experimental import pallas as pl


def kernel(x, table, W1, b1, W2, b2):
    raise NotImplementedError("write your pallas kernel here")


# speedup vs baseline: 9.5835x; 9.5835x over previous
import functools

import jax
import jax.numpy as jnp
from jax import lax
from jax.experimental import pallas as pl
from jax.experimental.pallas import tpu as pltpu
from jax.experimental.pallas import tpu_sc as plsc

B = 16384
L = 200
LP = 256
EMB = 16
NC = 2
NS = 16
NW = NC * NS
BPW = B // NW
BB = 16
NBLK = BPW // BB
RPB = BB * L


def _sc_pool_body(
    xp_hbm, table_hbm, out_hbm, idx0, idx1, rows0, rows1, accblk, sem0, sem1
):
    wid = lax.axis_index("c") * NS + lax.axis_index("s")

    def row_copies(idx_b, rows_b, sem_b):
        out = []
        for i in range(BB):
            out.append(
                pltpu.make_async_copy(
                    table_hbm.at[idx_b.at[i, pl.ds(0, 128)]],
                    rows_b.at[pl.ds(i * L, 128)],
                    sem_b,
                )
            )
            out.append(
                pltpu.make_async_copy(
                    table_hbm.at[idx_b.at[i, pl.ds(128, L - 128)]],
                    rows_b.at[pl.ds(i * L + 128, L - 128)],
                    sem_b,
                )
            )
        return out

    def stage_and_fire(blk, idx_b, rows_b, sem_b):
        pltpu.sync_copy(xp_hbm.at[pl.ds(wid * BPW + blk * BB, BB)], idx_b)
        for c in row_copies(idx_b, rows_b, sem_b):
            c.start()

    def drain_acc_store(blk, idx_b, rows_b, sem_b):
        for c in row_copies(idx_b, rows_b, sem_b):
            c.wait()

        zero = jnp.zeros((EMB,), jnp.float32)

        def acc_step(l, accs):
            return tuple(accs[i] + rows_b[i * L + l] for i in range(BB))

        accs = lax.fori_loop(0, L, acc_step, (zero,) * BB)
        for i in range(BB):
            accblk[i] = accs[i]
        pltpu.sync_copy(accblk, out_hbm.at[pl.ds(wid * BPW + blk * BB, BB)])

    stage_and_fire(0, idx0, rows0, sem0)

    def pair(g, carry):
        blk0 = 2 * g
        blk1 = blk0 + 1
        stage_and_fire(blk1, idx1, rows1, sem1)
        drain_acc_store(blk0, idx0, rows0, sem0)

        @pl.when(blk1 + 1 < NBLK)
        def _():
            stage_and_fire(blk1 + 1, idx0, rows0, sem0)

        drain_acc_store(blk1, idx1, rows1, sem1)
        return carry

    lax.fori_loop(0, NBLK // 2, pair, 0)


def _xpose_body(xt_ref, o_ref):
    xt = xt_ref[...].T
    pad = jnp.zeros((xt.shape[0], LP - L), jnp.int32)
    o_ref[...] = jnp.concatenate([xt, pad], axis=1)


def _mlp_body(s_ref, w1_ref, b1_ref, w2_ref, b2_ref, o_ref):
    h = jnp.tanh(s_ref[...] * (1.0 / L))
    h = jnp.tanh(jnp.dot(h, w1_ref[...]) + b1_ref[...])
    o_ref[...] = jnp.dot(h, w2_ref[...]) + b2_ref[...]


@jax.jit
def kernel(x, table, W1, b1, W2, b2):
    XB = 2048
    xp = pl.pallas_call(
        _xpose_body,
        grid=(B // XB,),
        in_specs=[pl.BlockSpec((L, XB), lambda i: (0, i))],
        out_specs=pl.BlockSpec((XB, LP), lambda i: (i, 0)),
        out_shape=jax.ShapeDtypeStruct((B, LP), jnp.int32),
    )(x.astype(jnp.int32).T)

    mesh = plsc.VectorSubcoreMesh(
        core_axis_name="c", subcore_axis_name="s", num_cores=NC, num_subcores=NS
    )
    sums = pl.kernel(
        _sc_pool_body,
        mesh=mesh,
        out_type=jax.ShapeDtypeStruct((B, EMB), jnp.float32),
        scratch_types=[
            pltpu.VMEM((BB, LP), jnp.int32),
            pltpu.VMEM((BB, LP), jnp.int32),
            pltpu.VMEM((RPB, EMB), jnp.float32),
            pltpu.VMEM((RPB, EMB), jnp.float32),
            pltpu.VMEM((BB, EMB), jnp.float32),
            pltpu.SemaphoreType.DMA,
            pltpu.SemaphoreType.DMA,
        ],
        compiler_params=pltpu.CompilerParams(use_tc_tiling_on_sc=False),
    )(xp, table)

    TBLK = 4096
    out = pl.pallas_call(
        _mlp_body,
        grid=(B // TBLK,),
        in_specs=[
            pl.BlockSpec((TBLK, EMB), lambda i: (i, 0)),
            pl.BlockSpec((EMB, EMB), lambda i: (0, 0)),
            pl.BlockSpec((1, EMB), lambda i: (0, 0)),
            pl.BlockSpec((EMB, 2), lambda i: (0, 0)),
            pl.BlockSpec((1, 2), lambda i: (0, 0)),
        ],
        out_specs=pl.BlockSpec((TBLK, 2), lambda i: (i, 0)),
        out_shape=jax.ShapeDtypeStruct((B, 2), jnp.float32),
    )(sums, W1, b1.reshape(1, EMB), W2, b2.reshape(1, 2))
    return out


# speedup vs baseline: 13.2683x; 1.3845x over previous
import functools

import jax
import jax.numpy as jnp
from jax import lax
from jax.experimental import pallas as pl
from jax.experimental.pallas import tpu as pltpu
from jax.experimental.pallas import tpu_sc as plsc

B = 16384
L = 200
EMB = 16
V = 1000000
NC = 2
NS = 16
NW = NC * NS
BPW = B // NW
BB = 16
NBLK = BPW // BB
RPB = BB * L

TCH = 8192
G = TCH // 8
NTB = (V + TCH - 1) // TCH
LAST = (NTB - 1) * TCH
VP = NTB * TCH


def _remap(r):
    i = r & ~(TCH - 1)
    loc = r & (TCH - 1)
    k = loc >> 10
    g = loc & (G - 1)
    rho_main = i | (g << 3) | k
    rho_tail = LAST + ((r - LAST) << 3)
    return jnp.where(r < LAST, rho_main, rho_tail)


def _xprep_body(x_ref, lo_ref, hi_ref):
    t = _remap(x_ref[...]).T
    lo_ref[...] = t[:, :128]
    hi_ref[...] = jnp.concatenate(
        [t[:, 128:L], jnp.zeros((t.shape[0], 256 - L), jnp.int32)], axis=1
    )


def _pack_body(t_ref, o_ref):
    tr = t_ref[...].T
    parts = [tr[k * G:(k + 1) * G, :] for k in range(8)]
    o_ref[...] = jnp.concatenate(parts, axis=1)


def _sc_pool_body(
    xlo_hbm, xhi_hbm, table_hbm, out_hbm,
    ilo0, ihi0, ilo1, ihi1, rows0, rows1, accblk, sem0, sem1,
):
    wid = lax.axis_index("c") * NS + lax.axis_index("s")

    def row_copies(ilo_b, ihi_b, rows_b, sem_b):
        out = []
        for i in range(BB):
            out.append(
                pltpu.make_async_copy(
                    table_hbm.at[ilo_b.at[i]],
                    rows_b.at[pl.ds(i * L, 128)],
                    sem_b,
                )
            )
            out.append(
                pltpu.make_async_copy(
                    table_hbm.at[ihi_b.at[i, pl.ds(0, L - 128)]],
                    rows_b.at[pl.ds(i * L + 128, L - 128)],
                    sem_b,
                )
            )
        return out

    def stage_and_fire(blk, ilo_b, ihi_b, rows_b, sem_b):
        s0 = wid * BPW + blk * BB
        pltpu.sync_copy(xlo_hbm.at[pl.ds(s0, BB)], ilo_b)
        pltpu.sync_copy(xhi_hbm.at[pl.ds(s0, BB)], ihi_b)
        for c in row_copies(ilo_b, ihi_b, rows_b, sem_b):
            c.start()

    def drain_acc_store(blk, ilo_b, ihi_b, rows_b, sem_b):
        for c in row_copies(ilo_b, ihi_b, rows_b, sem_b):
            c.wait()

        zero = jnp.zeros((EMB,), jnp.float32)

        def acc_step(l, accs):
            return tuple(accs[i] + rows_b[i * L + l] for i in range(BB))

        accs = lax.fori_loop(0, L, acc_step, (zero,) * BB)
        for i in range(BB):
            accblk[i] = accs[i]
        pltpu.sync_copy(accblk, out_hbm.at[pl.ds(wid * BPW + blk * BB, BB)])

    stage_and_fire(0, ilo0, ihi0, rows0, sem0)

    def pair(g, carry):
        blk0 = 2 * g
        blk1 = blk0 + 1
        stage_and_fire(blk1, ilo1, ihi1, rows1, sem1)
        drain_acc_store(blk0, ilo0, ihi0, rows0, sem0)

        @pl.when(blk1 + 1 < NBLK)
        def _():
            stage_and_fire(blk1 + 1, ilo0, ihi0, rows0, sem0)

        drain_acc_store(blk1, ilo1, ihi1, rows1, sem1)
        return carry

    lax.fori_loop(0, NBLK // 2, pair, 0)


def _mlp_body(s_ref, w1_ref, b1_ref, w2_ref, b2_ref, o_ref):
    h = jnp.tanh(s_ref[...] * (1.0 / L))
    h = jnp.tanh(jnp.dot(h, w1_ref[...]) + b1_ref[...])
    o_ref[...] = jnp.dot(h, w2_ref[...]) + b2_ref[...]


@jax.jit
def kernel(x, table, W1, b1, W2, b2):
    XB = 2048
    xlo, xhi = pl.pallas_call(
        _xprep_body,
        grid=(B // XB,),
        in_specs=[pl.BlockSpec((L, XB), lambda i: (0, i))],
        out_specs=[
            pl.BlockSpec((XB, 128), lambda i: (i, 0)),
            pl.BlockSpec((XB, 128), lambda i: (i, 0)),
        ],
        out_shape=[
            jax.ShapeDtypeStruct((B, 128), jnp.int32),
            jax.ShapeDtypeStruct((B, 128), jnp.int32),
        ],
    )(x.astype(jnp.int32).T)

    table_pk = pl.pallas_call(
        _pack_body,
        grid=(NTB,),
        in_specs=[pl.BlockSpec((EMB, TCH), lambda i: (0, i))],
        out_specs=pl.BlockSpec((G, 128), lambda i: (i, 0)),
        out_shape=jax.ShapeDtypeStruct((NTB * G, 128), jnp.float32),
    )(table.T)
    table_l = table_pk.reshape(VP, EMB)

    mesh = plsc.VectorSubcoreMesh(
        core_axis_name="c", subcore_axis_name="s", num_cores=NC, num_subcores=NS
    )
    sums = pl.kernel(
        _sc_pool_body,
        mesh=mesh,
        out_type=jax.ShapeDtypeStruct((B, EMB), jnp.float32),
        scratch_types=[
            pltpu.VMEM((BB, 128), jnp.int32),
            pltpu.VMEM((BB, 128), jnp.int32),
            pltpu.VMEM((BB, 128), jnp.int32),
            pltpu.VMEM((BB, 128), jnp.int32),
            pltpu.VMEM((RPB, EMB), jnp.float32),
            pltpu.VMEM((RPB, EMB), jnp.float32),
            pltpu.VMEM((BB, EMB), jnp.float32),
            pltpu.SemaphoreType.DMA,
            pltpu.SemaphoreType.DMA,
        ],
        compiler_params=pltpu.CompilerParams(use_tc_tiling_on_sc=False),
    )(xlo, xhi, table_l)

    TBLK = 4096
    out = pl.pallas_call(
        _mlp_body,
        grid=(B // TBLK,),
        in_specs=[
            pl.BlockSpec((TBLK, EMB), lambda i: (i, 0)),
            pl.BlockSpec((EMB, EMB), lambda i: (0, 0)),
            pl.BlockSpec((1, EMB), lambda i: (0, 0)),
            pl.BlockSpec((EMB, 2), lambda i: (0, 0)),
            pl.BlockSpec((1, 2), lambda i: (0, 0)),
        ],
        out_specs=pl.BlockSpec((TBLK, 2), lambda i: (i, 0)),
        out_shape=jax.ShapeDtypeStruct((B, 2), jnp.float32),
    )(sums, W1, b1.reshape(1, EMB), W2, b2.reshape(1, 2))
    return out


# speedup vs baseline: 21.5766x; 1.6262x over previous
import functools

import jax
import jax.numpy as jnp
from jax import lax
from jax.experimental import pallas as pl
from jax.experimental.pallas import tpu as pltpu
from jax.experimental.pallas import tpu_sc as plsc

B = 16384
L = 200
EMB = 16
V = 1000000
NC = 2
NS = 16
NW = NC * NS
BPW = B // NW
BB = 16
NBLK = BPW // BB
RPB = BB * L

TCH = 8192
G = TCH // 8
NTB = (V + TCH - 1) // TCH
LAST = (NTB - 1) * TCH
VP = NTB * TCH


def _remap(r):
    i = r & ~(TCH - 1)
    loc = r & (TCH - 1)
    k = loc >> 10
    g = loc & (G - 1)
    rho_main = i | (g << 3) | k
    rho_tail = LAST + ((r - LAST) << 3)
    return jnp.where(r < LAST, rho_main, rho_tail)


def _xprep_body(x_ref, lo_ref, hi_ref):
    t = _remap(x_ref[...]).T
    lo_ref[...] = t[:, :128]
    hi_ref[...] = jnp.concatenate(
        [t[:, 128:L], jnp.zeros((t.shape[0], 256 - L), jnp.int32)], axis=1
    )


def _pack_body(t_ref, o_ref):
    qq = jnp.concatenate([t_ref[:, k * G:(k + 1) * G] for k in range(8)], axis=0)
    o_ref[...] = qq.T


def _sc_pool_body(
    xlo_hbm, xhi_hbm, table_hbm, out_hbm,
    ilo0, ihi0, ilo1, ihi1, rows0, rows1, accblk, sem0, sem1, ssem0, ssem1,
):
    wid = lax.axis_index("c") * NS + lax.axis_index("s")

    def stage_copies(blk, ilo_b, ihi_b, ssem_b):
        s0 = wid * BPW + blk * BB
        return [
            pltpu.make_async_copy(xlo_hbm.at[pl.ds(s0, BB)], ilo_b, ssem_b),
            pltpu.make_async_copy(xhi_hbm.at[pl.ds(s0, BB)], ihi_b, ssem_b),
        ]

    def stage_start(blk, ilo_b, ihi_b, ssem_b):
        for c in stage_copies(blk, ilo_b, ihi_b, ssem_b):
            c.start()

    def stage_wait(blk, ilo_b, ihi_b, ssem_b):
        for c in stage_copies(blk, ilo_b, ihi_b, ssem_b):
            c.wait()

    def row_copies(ilo_b, ihi_b, rows_b, sem_b):
        out = []
        for i in range(BB):
            out.append(
                pltpu.make_async_copy(
                    table_hbm.at[ilo_b.at[i]],
                    rows_b.at[pl.ds(i * L, 128)],
                    sem_b,
                )
            )
            out.append(
                pltpu.make_async_copy(
                    table_hbm.at[ihi_b.at[i, pl.ds(0, L - 128)]],
                    rows_b.at[pl.ds(i * L + 128, L - 128)],
                    sem_b,
                )
            )
        return out

    def fire(ilo_b, ihi_b, rows_b, sem_b):
        for c in row_copies(ilo_b, ihi_b, rows_b, sem_b):
            c.start()

    def drain(ilo_b, ihi_b, rows_b, sem_b):
        for c in row_copies(ilo_b, ihi_b, rows_b, sem_b):
            c.wait()

    def acc_store(blk, rows_b):
        zero = jnp.zeros((EMB,), jnp.float32)

        def acc_step(l, accs):
            return tuple(accs[i] + rows_b[i * L + l] for i in range(BB))

        accs = lax.fori_loop(0, L, acc_step, (zero,) * BB)
        for i in range(BB):
            accblk[i] = accs[i]
        pltpu.sync_copy(accblk, out_hbm.at[pl.ds(wid * BPW + blk * BB, BB)])

    stage_start(0, ilo0, ihi0, ssem0)
    stage_wait(0, ilo0, ihi0, ssem0)
    fire(ilo0, ihi0, rows0, sem0)
    stage_start(1, ilo1, ihi1, ssem1)

    def pair(g, carry):
        blk0 = 2 * g
        blk1 = blk0 + 1
        stage_wait(blk1, ilo1, ihi1, ssem1)
        fire(ilo1, ihi1, rows1, sem1)
        drain(ilo0, ihi0, rows0, sem0)

        @pl.when(blk0 + 2 < NBLK)
        def _():
            stage_start(blk0 + 2, ilo0, ihi0, ssem0)

        acc_store(blk0, rows0)

        @pl.when(blk0 + 2 < NBLK)
        def _():
            stage_wait(blk0 + 2, ilo0, ihi0, ssem0)
            fire(ilo0, ihi0, rows0, sem0)

        drain(ilo1, ihi1, rows1, sem1)

        @pl.when(blk1 + 2 < NBLK)
        def _():
            stage_start(blk1 + 2, ilo1, ihi1, ssem1)

        acc_store(blk1, rows1)
        return carry

    lax.fori_loop(0, NBLK // 2, pair, 0)


def _mlp_body(s_ref, w1_ref, b1_ref, w2_ref, b2_ref, o_ref):
    h = jnp.tanh(s_ref[...] * (1.0 / L))
    h = jnp.tanh(jnp.dot(h, w1_ref[...]) + b1_ref[...])
    o_ref[...] = jnp.dot(h, w2_ref[...]) + b2_ref[...]


@jax.jit
def kernel(x, table, W1, b1, W2, b2):
    XB = 2048
    xlo, xhi = pl.pallas_call(
        _xprep_body,
        grid=(B // XB,),
        in_specs=[pl.BlockSpec((L, XB), lambda i: (0, i))],
        out_specs=[
            pl.BlockSpec((XB, 128), lambda i: (i, 0)),
            pl.BlockSpec((XB, 128), lambda i: (i, 0)),
        ],
        out_shape=[
            jax.ShapeDtypeStruct((B, 128), jnp.int32),
            jax.ShapeDtypeStruct((B, 128), jnp.int32),
        ],
    )(x.astype(jnp.int32).T)

    table_pk = pl.pallas_call(
        _pack_body,
        grid=(NTB,),
        in_specs=[pl.BlockSpec((EMB, TCH), lambda i: (0, i))],
        out_specs=pl.BlockSpec((G, 128), lambda i: (i, 0)),
        out_shape=jax.ShapeDtypeStruct((NTB * G, 128), jnp.float32),
    )(table.T)
    table_l = table_pk.reshape(VP, EMB)

    mesh = plsc.VectorSubcoreMesh(
        core_axis_name="c", subcore_axis_name="s", num_cores=NC, num_subcores=NS
    )
    sums = pl.kernel(
        _sc_pool_body,
        mesh=mesh,
        out_type=jax.ShapeDtypeStruct((B, EMB), jnp.float32),
        scratch_types=[
            pltpu.VMEM((BB, 128), jnp.int32),
            pltpu.VMEM((BB, 128), jnp.int32),
            pltpu.VMEM((BB, 128), jnp.int32),
            pltpu.VMEM((BB, 128), jnp.int32),
            pltpu.VMEM((RPB, EMB), jnp.float32),
            pltpu.VMEM((RPB, EMB), jnp.float32),
            pltpu.VMEM((BB, EMB), jnp.float32),
            pltpu.SemaphoreType.DMA,
            pltpu.SemaphoreType.DMA,
            pltpu.SemaphoreType.DMA,
            pltpu.SemaphoreType.DMA,
        ],
        compiler_params=pltpu.CompilerParams(use_tc_tiling_on_sc=False),
    )(xlo, xhi, table_l)

    TBLK = 4096
    out = pl.pallas_call(
        _mlp_body,
        grid=(B // TBLK,),
        in_specs=[
            pl.BlockSpec((TBLK, EMB), lambda i: (i, 0)),
            pl.BlockSpec((EMB, EMB), lambda i: (0, 0)),
            pl.BlockSpec((1, EMB), lambda i: (0, 0)),
            pl.BlockSpec((EMB, 2), lambda i: (0, 0)),
            pl.BlockSpec((1, 2), lambda i: (0, 0)),
        ],
        out_specs=pl.BlockSpec((TBLK, 2), lambda i: (i, 0)),
        out_shape=jax.ShapeDtypeStruct((B, 2), jnp.float32),
    )(sums, W1, b1.reshape(1, EMB), W2, b2.reshape(1, 2))
    return out


# speedup vs baseline: 25.1804x; 1.1670x over previous
import jax
import jax.numpy as jnp
from jax import lax
from jax.experimental import pallas as pl
from jax.experimental.pallas import tpu as pltpu
from jax.experimental.pallas import tpu_sc as plsc

B = 16384
L = 200
EMB = 16
V = 1000000
NC = 2
NS = 16
NW = NC * NS
BPW = B // NW
BB = 16
NBLK = BPW // BB
RPB = BB * L

TCH = 8192
G = TCH // 8
NTB = (V + TCH - 1) // TCH
LAST = (NTB - 1) * TCH
VP = NTB * TCH
PCH = 3


def _remap(r):
    i = r & ~(TCH - 1)
    loc = r & (TCH - 1)
    k = loc >> 10
    g = loc & (G - 1)
    rho_main = i | (g << 3) | k
    rho_tail = LAST + ((r - LAST) << 3)
    return jnp.where(r < LAST, rho_main, rho_tail)


def _xprep_body(x_ref, lo_ref, hi_ref):
    t = _remap(x_ref[...]).T
    lo_ref[...] = t[:, :128]
    hi_ref[...] = jnp.concatenate(
        [t[:, 128:L], jnp.zeros((t.shape[0], 256 - L), jnp.int32)], axis=1
    )


def _pack_body(t_ref, o_ref):
    for j in range(PCH):
        base = j * TCH
        qq = jnp.concatenate(
            [t_ref[:, base + k * G:base + (k + 1) * G] for k in range(8)], axis=0
        )
        o_ref[pl.ds(j * G, G), :] = qq.T


def _sc_pool_body(
    xlo_hbm, xhi_hbm, table_hbm, out_hbm,
    ilo0, ihi0, ilo1, ihi1, rows0, rows1, accblk, sem0, sem1, ssem0, ssem1,
):
    wid = lax.axis_index("c") * NS + lax.axis_index("s")

    def stage_copies(blk, ilo_b, ihi_b, ssem_b):
        s0 = wid * BPW + blk * BB
        return [
            pltpu.make_async_copy(xlo_hbm.at[pl.ds(s0, BB)], ilo_b, ssem_b),
            pltpu.make_async_copy(xhi_hbm.at[pl.ds(s0, BB)], ihi_b, ssem_b),
        ]

    def stage_start(blk, ilo_b, ihi_b, ssem_b):
        for c in stage_copies(blk, ilo_b, ihi_b, ssem_b):
            c.start()

    def stage_wait(blk, ilo_b, ihi_b, ssem_b):
        for c in stage_copies(blk, ilo_b, ihi_b, ssem_b):
            c.wait()

    def row_copies(ilo_b, ihi_b, rows_b, sem_b):
        out = []
        for i in range(BB):
            out.append(
                pltpu.make_async_copy(
                    table_hbm.at[ilo_b.at[i]],
                    rows_b.at[pl.ds(i * L, 128)],
                    sem_b,
                )
            )
            out.append(
                pltpu.make_async_copy(
                    table_hbm.at[ihi_b.at[i, pl.ds(0, L - 128)]],
                    rows_b.at[pl.ds(i * L + 128, L - 128)],
                    sem_b,
                )
            )
        return out

    def fire(ilo_b, ihi_b, rows_b, sem_b):
        for c in row_copies(ilo_b, ihi_b, rows_b, sem_b):
            c.start()

    def drain(ilo_b, ihi_b, rows_b, sem_b):
        for c in row_copies(ilo_b, ihi_b, rows_b, sem_b):
            c.wait()

    def acc_store(blk, rows_b):
        zero = jnp.zeros((EMB,), jnp.float32)

        def acc_step(l, accs):
            return tuple(accs[i] + rows_b[i * L + l] for i in range(BB))

        accs = lax.fori_loop(0, L, acc_step, (zero,) * BB)
        for i in range(BB):
            accblk[i] = accs[i]
        pltpu.sync_copy(accblk, out_hbm.at[pl.ds(wid * BPW + blk * BB, BB)])

    stage_start(0, ilo0, ihi0, ssem0)
    stage_wait(0, ilo0, ihi0, ssem0)
    fire(ilo0, ihi0, rows0, sem0)
    stage_start(1, ilo1, ihi1, ssem1)

    def pair(g, carry):
        blk0 = 2 * g
        blk1 = blk0 + 1
        stage_wait(blk1, ilo1, ihi1, ssem1)
        fire(ilo1, ihi1, rows1, sem1)
        drain(ilo0, ihi0, rows0, sem0)

        @pl.when(blk0 + 2 < NBLK)
        def _():
            stage_start(blk0 + 2, ilo0, ihi0, ssem0)

        acc_store(blk0, rows0)

        @pl.when(blk0 + 2 < NBLK)
        def _():
            stage_wait(blk0 + 2, ilo0, ihi0, ssem0)
            fire(ilo0, ihi0, rows0, sem0)

        drain(ilo1, ihi1, rows1, sem1)

        @pl.when(blk1 + 2 < NBLK)
        def _():
            stage_start(blk1 + 2, ilo1, ihi1, ssem1)

        acc_store(blk1, rows1)
        return carry

    lax.fori_loop(0, NBLK // 2, pair, 0)


def _mlp_body(s_ref, w1_ref, b1_ref, w2_ref, b2_ref, o_ref):
    h = jnp.tanh(s_ref[...] * (1.0 / L))
    h = jnp.tanh(jnp.dot(h, w1_ref[...]) + b1_ref[...])
    o_ref[...] = jnp.dot(h, w2_ref[...]) + b2_ref[...]


@jax.jit
def kernel(x, table, W1, b1, W2, b2):
    XB = 2048
    xlo, xhi = pl.pallas_call(
        _xprep_body,
        grid=(B // XB,),
        in_specs=[pl.BlockSpec((L, XB), lambda i: (0, i))],
        out_specs=[
            pl.BlockSpec((XB, 128), lambda i: (i, 0)),
            pl.BlockSpec((XB, 128), lambda i: (i, 0)),
        ],
        out_shape=[
            jax.ShapeDtypeStruct((B, 128), jnp.int32),
            jax.ShapeDtypeStruct((B, 128), jnp.int32),
        ],
    )(x.astype(jnp.int32).T)

    table_pk = pl.pallas_call(
        _pack_body,
        grid=(NTB // PCH,),
        in_specs=[pl.BlockSpec((EMB, PCH * TCH), lambda i: (0, i))],
        out_specs=pl.BlockSpec((PCH * G, 128), lambda i: (i, 0)),
        out_shape=jax.ShapeDtypeStruct((NTB * G, 128), jnp.float32),
    )(table.T)
    table_l = table_pk.reshape(VP, EMB)

    mesh = plsc.VectorSubcoreMesh(
        core_axis_name="c", subcore_axis_name="s", num_cores=NC, num_subcores=NS
    )
    sums = pl.kernel(
        _sc_pool_body,
        mesh=mesh,
        out_type=jax.ShapeDtypeStruct((B, EMB), jnp.float32),
        scratch_types=[
            pltpu.VMEM((BB, 128), jnp.int32),
            pltpu.VMEM((BB, 128), jnp.int32),
            pltpu.VMEM((BB, 128), jnp.int32),
            pltpu.VMEM((BB, 128), jnp.int32),
            pltpu.VMEM((RPB, EMB), jnp.float32),
            pltpu.VMEM((RPB, EMB), jnp.float32),
            pltpu.VMEM((BB, EMB), jnp.float32),
            pltpu.SemaphoreType.DMA,
            pltpu.SemaphoreType.DMA,
            pltpu.SemaphoreType.DMA,
            pltpu.SemaphoreType.DMA,
        ],
        compiler_params=pltpu.CompilerParams(use_tc_tiling_on_sc=False),
    )(xlo, xhi, table_l)

    TBLK = 16384
    out = pl.pallas_call(
        _mlp_body,
        grid=(B // TBLK,),
        in_specs=[
            pl.BlockSpec((TBLK, EMB), lambda i: (i, 0)),
            pl.BlockSpec((EMB, EMB), lambda i: (0, 0)),
            pl.BlockSpec((1, EMB), lambda i: (0, 0)),
            pl.BlockSpec((EMB, 2), lambda i: (0, 0)),
            pl.BlockSpec((1, 2), lambda i: (0, 0)),
        ],
        out_specs=pl.BlockSpec((TBLK, 2), lambda i: (i, 0)),
        out_shape=jax.ShapeDtypeStruct((B, 2), jnp.float32),
    )(sums, W1, b1.reshape(1, EMB), W2, b2.reshape(1, 2))
    return out
